# baseline (device time: 40901 ns/iter reference)
import jax
import jax.numpy as jnp
from jax import lax
from jax.experimental import pallas as pl
from jax.experimental.pallas import tpu as pltpu

N_DEV = 4
DH = 64
BLK = 64


def _fused_attn_allreduce(x_f32, wq_bf, k_f32, v_f32, wo_bf, bias, B, Sq):
    m, e = x_f32.shape
    hq = wq_bf.shape[1] // DH
    skv = k_f32.shape[1]
    c = m // N_DEV
    cpb = (m // B) // c

    def chunk_partial(ck, x_ref, wq_ref, kv_bf, wo_ref, bias_ref):
        b = ck // cpb
        qr = lax.rem(ck, cpb) * c
        xq = x_ref[pl.ds(ck * c, c), :].astype(jnp.bfloat16)
        q_all = jnp.dot(
            xq, wq_ref[:, :], preferred_element_type=jnp.float32,
        ).astype(jnp.bfloat16)
        bias_q = bias_ref[pl.ds(qr, c), :]
        k_b = kv_bf[0, b]
        v_b = kv_bf[1, b]
        part = jnp.zeros((c, e), dtype=jnp.float32)
        for h in range(hq):
            q_h = q_all[:, h * DH:(h + 1) * DH]
            k_h = k_b[:, h * DH:(h + 1) * DH]
            s_h = lax.dot_general(
                q_h, k_h, (((1,), (1,)), ((), ())),
                preferred_element_type=jnp.float32,
            ) * 0.125 + bias_q
            w_h = jnp.exp(s_h)
            denom = jnp.sum(w_h, axis=1, keepdims=True)
            ctx_h = jnp.dot(
                w_h.astype(jnp.bfloat16), v_b[:, h * DH:(h + 1) * DH],
                preferred_element_type=jnp.float32,
            ) / denom
            part = part + jnp.dot(
                ctx_h.astype(jnp.bfloat16),
                wo_ref[h * DH:(h + 1) * DH, :],
                preferred_element_type=jnp.float32,
            )
        return part

    def body(x_ref, wq_ref, kf_ref, vf_ref, wo_ref, bias_ref, out_ref,
             kv_bf, send_buf, rs_buf, ag_buf, red_bf,
             rs_send_sems, rs_recv_sems, ag_send_sems, ag_recv_sems):
        my = lax.axis_index("i")

        for b in range(B):
            kv_bf[0, b, :, :] = kf_ref[b].astype(jnp.bfloat16)
            kv_bf[1, b, :, :] = vf_ref[b].astype(jnp.bfloat16)

        barrier_sem = pltpu.get_barrier_semaphore()
        for o in (1, 2, 3):
            pl.semaphore_signal(
                barrier_sem, inc=1,
                device_id=(lax.rem(my + o, N_DEV),),
                device_id_type=pl.DeviceIdType.MESH,
            )
        pl.semaphore_wait(barrier_sem, 3)

        rs_sends = []
        for o in (1, 2, 3):
            peer = lax.rem(my + o, N_DEV)
            part = chunk_partial(
                peer, x_ref, wq_ref, kv_bf, wo_ref, bias_ref)
            send_buf[o - 1, :, :] = part.astype(jnp.bfloat16)
            rdma = pltpu.make_async_remote_copy(
                src_ref=send_buf.at[o - 1],
                dst_ref=rs_buf.at[3 - o],
                send_sem=rs_send_sems.at[o - 1],
                recv_sem=rs_recv_sems.at[3 - o],
                device_id=(peer,),
                device_id_type=pl.DeviceIdType.MESH,
            )
            rdma.start()
            rs_sends.append(rdma)

        acc = chunk_partial(
            my, x_ref, wq_ref, kv_bf, wo_ref, bias_ref)
        for slot in range(3):
            recv = pltpu.make_async_remote_copy(
                src_ref=rs_buf.at[slot],
                dst_ref=rs_buf.at[slot],
                send_sem=rs_send_sems.at[slot],
                recv_sem=rs_recv_sems.at[slot],
                device_id=(my,),
                device_id_type=pl.DeviceIdType.MESH,
            )
            recv.wait_recv()
            acc = acc + rs_buf[slot, :, :].astype(jnp.float32)

        out_ref[pl.ds(my * c, c), :] = acc
        red_bf[:, :] = acc.astype(jnp.bfloat16)
        for rdma in rs_sends:
            rdma.wait_send()

        ag_sends = []
        for o in (1, 2, 3):
            peer = lax.rem(my + o, N_DEV)
            rdma = pltpu.make_async_remote_copy(
                src_ref=red_bf,
                dst_ref=ag_buf.at[3 - o],
                send_sem=ag_send_sems.at[o - 1],
                recv_sem=ag_recv_sems.at[3 - o],
                device_id=(peer,),
                device_id_type=pl.DeviceIdType.MESH,
            )
            rdma.start()
            ag_sends.append(rdma)

        for slot in range(3):
            recv = pltpu.make_async_remote_copy(
                src_ref=ag_buf.at[slot],
                dst_ref=ag_buf.at[slot],
                send_sem=ag_send_sems.at[slot],
                recv_sem=ag_recv_sems.at[slot],
                device_id=(my,),
                device_id_type=pl.DeviceIdType.MESH,
            )
            recv.wait_recv()
            peer = lax.rem(my + slot + 1, N_DEV)
            out_ref[pl.ds(peer * c, c), :] = ag_buf[slot, :, :].astype(
                jnp.float32
            )
        for rdma in ag_sends:
            rdma.wait_send()

    return pl.pallas_call(
        body,
        out_shape=jax.ShapeDtypeStruct((m, e), jnp.float32),
        in_specs=[pl.BlockSpec(memory_space=pltpu.VMEM)] * 6,
        out_specs=pl.BlockSpec(memory_space=pltpu.VMEM),
        scratch_shapes=[
            pltpu.VMEM((2, B, skv, hq * DH), jnp.bfloat16),
            pltpu.VMEM((3, c, e), jnp.bfloat16),
            pltpu.VMEM((3, c, e), jnp.bfloat16),
            pltpu.VMEM((3, c, e), jnp.bfloat16),
            pltpu.VMEM((c, e), jnp.bfloat16),
            pltpu.SemaphoreType.DMA((3,)),
            pltpu.SemaphoreType.DMA((3,)),
            pltpu.SemaphoreType.DMA((3,)),
            pltpu.SemaphoreType.DMA((3,)),
        ],
        compiler_params=pltpu.CompilerParams(collective_id=0),
    )(x_f32, wq_bf, k_f32, v_f32, wo_bf, bias)


def kernel(x, Wq, K_ext, V_ext, Wo):
    B, Sq, E = x.shape
    hq = Wq.shape[1] // DH
    Skv = K_ext.shape[1]
    my = lax.axis_index("i")

    bf = jnp.bfloat16
    K = lax.dynamic_slice_in_dim(K_ext, my * hq, hq, axis=2)
    V = lax.dynamic_slice_in_dim(V_ext, my * hq, hq, axis=2)
    Kn = K.reshape(B, Skv, hq * DH)
    Vn = V.reshape(B, Skv, hq * DH)

    import numpy as np
    qb = (np.arange(Sq) // BLK)[:, None]
    kb = (np.arange(Skv) // BLK)[None, :]
    mask = (qb == kb) | (kb == 0) | ((qb + kb) % 3 == 0)
    bias = jnp.asarray(np.where(mask, 0.0, -1e9).astype(np.float32))

    out = _fused_attn_allreduce(
        x.reshape(B * Sq, E), Wq.astype(bf),
        Kn, Vn, Wo.astype(bf), bias, B, Sq,
    )
    return out.reshape(B, Sq, E)


# device time: 36442 ns/iter; 1.1224x vs baseline; 1.1224x over previous
import jax
import jax.numpy as jnp
from jax import lax
from jax.experimental import pallas as pl
from jax.experimental.pallas import tpu as pltpu

N_DEV = 4
DH = 64
BLK = 64


def _fused_attn_allreduce(x_bf, wq_bf, kh_bf, vh_bf, wo_bf, bias, B, Sq):
    m, e = x_bf.shape
    hq = wq_bf.shape[1] // DH
    skv = kh_bf.shape[1]
    c = m // N_DEV
    cpb = (m // B) // c

    def chunk_partial(ck, x_ref, wq_ref, kh_ref, vh_ref, wo_ref, bias_ref):
        b = ck // cpb
        qr = lax.rem(ck, cpb) * c
        xq = x_ref[pl.ds(ck * c, c), :].astype(jnp.bfloat16)
        q_all = jnp.dot(
            xq, wq_ref[:, :], preferred_element_type=jnp.float32,
        ).astype(jnp.bfloat16)
        bias_q = bias_ref[pl.ds(qr, c), :]
        ctxs = []
        for h in range(hq):
            idx = b * hq + h
            q_h = q_all[:, h * DH:(h + 1) * DH]
            k_h = kh_ref[idx]
            s_h = lax.dot_general(
                q_h, k_h, (((1,), (1,)), ((), ())),
                preferred_element_type=jnp.float32,
            ) + bias_q
            w_h = jnp.exp(s_h)
            denom = jnp.sum(w_h, axis=1, keepdims=True)
            ctx_h = jnp.dot(
                w_h.astype(jnp.bfloat16), vh_ref[idx],
                preferred_element_type=jnp.float32,
            ) / denom
            ctxs.append(ctx_h.astype(jnp.bfloat16))
        ctx_all = jnp.concatenate(ctxs, axis=1)
        return jnp.dot(
            ctx_all, wo_ref[:, :], preferred_element_type=jnp.float32,
        )

    def body(x_ref, wq_ref, kh_ref, vh_ref, wo_ref, bias_ref, out_ref,
             send_buf, rs_buf, ag_buf, red_bf,
             rs_send_sems, rs_recv_sems, ag_send_sems, ag_recv_sems):
        my = lax.axis_index("i")

        barrier_sem = pltpu.get_barrier_semaphore()
        for o in (1, 2, 3):
            pl.semaphore_signal(
                barrier_sem, inc=1,
                device_id=(lax.rem(my + o, N_DEV),),
                device_id_type=pl.DeviceIdType.MESH,
            )
        pl.semaphore_wait(barrier_sem, 3)

        rs_sends = []
        for o in (1, 2, 3):
            peer = lax.rem(my + o, N_DEV)
            part = chunk_partial(
                peer, x_ref, wq_ref, kh_ref, vh_ref, wo_ref, bias_ref)
            send_buf[o - 1, :, :] = part.astype(jnp.bfloat16)
            rdma = pltpu.make_async_remote_copy(
                src_ref=send_buf.at[o - 1],
                dst_ref=rs_buf.at[3 - o],
                send_sem=rs_send_sems.at[o - 1],
                recv_sem=rs_recv_sems.at[3 - o],
                device_id=(peer,),
                device_id_type=pl.DeviceIdType.MESH,
            )
            rdma.start()
            rs_sends.append(rdma)

        acc = chunk_partial(
            my, x_ref, wq_ref, kh_ref, vh_ref, wo_ref, bias_ref)
        for slot in range(3):
            recv = pltpu.make_async_remote_copy(
                src_ref=rs_buf.at[slot],
                dst_ref=rs_buf.at[slot],
                send_sem=rs_send_sems.at[slot],
                recv_sem=rs_recv_sems.at[slot],
                device_id=(my,),
                device_id_type=pl.DeviceIdType.MESH,
            )
            recv.wait_recv()
            acc = acc + rs_buf[slot, :, :].astype(jnp.float32)

        out_ref[pl.ds(my * c, c), :] = acc
        red_bf[:, :] = acc.astype(jnp.bfloat16)
        for rdma in rs_sends:
            rdma.wait_send()

        ag_sends = []
        for o in (1, 2, 3):
            peer = lax.rem(my + o, N_DEV)
            rdma = pltpu.make_async_remote_copy(
                src_ref=red_bf,
                dst_ref=ag_buf.at[3 - o],
                send_sem=ag_send_sems.at[o - 1],
                recv_sem=ag_recv_sems.at[3 - o],
                device_id=(peer,),
                device_id_type=pl.DeviceIdType.MESH,
            )
            rdma.start()
            ag_sends.append(rdma)

        for slot in range(3):
            recv = pltpu.make_async_remote_copy(
                src_ref=ag_buf.at[slot],
                dst_ref=ag_buf.at[slot],
                send_sem=ag_send_sems.at[slot],
                recv_sem=ag_recv_sems.at[slot],
                device_id=(my,),
                device_id_type=pl.DeviceIdType.MESH,
            )
            recv.wait_recv()
            peer = lax.rem(my + slot + 1, N_DEV)
            out_ref[pl.ds(peer * c, c), :] = ag_buf[slot, :, :].astype(
                jnp.float32
            )
        for rdma in ag_sends:
            rdma.wait_send()

    return pl.pallas_call(
        body,
        out_shape=jax.ShapeDtypeStruct((m, e), jnp.float32),
        in_specs=[pl.BlockSpec(memory_space=pltpu.VMEM)] * 6,
        out_specs=pl.BlockSpec(memory_space=pltpu.VMEM),
        scratch_shapes=[
            pltpu.VMEM((3, c, e), jnp.bfloat16),
            pltpu.VMEM((3, c, e), jnp.bfloat16),
            pltpu.VMEM((3, c, e), jnp.bfloat16),
            pltpu.VMEM((c, e), jnp.bfloat16),
            pltpu.SemaphoreType.DMA((3,)),
            pltpu.SemaphoreType.DMA((3,)),
            pltpu.SemaphoreType.DMA((3,)),
            pltpu.SemaphoreType.DMA((3,)),
        ],
        compiler_params=pltpu.CompilerParams(collective_id=0),
    )(x_bf, wq_bf, kh_bf, vh_bf, wo_bf, bias)


def kernel(x, Wq, K_ext, V_ext, Wo):
    B, Sq, E = x.shape
    hq = Wq.shape[1] // DH
    Skv = K_ext.shape[1]
    my = lax.axis_index("i")

    bf = jnp.bfloat16
    K = lax.dynamic_slice_in_dim(K_ext, my * hq, hq, axis=2)
    V = lax.dynamic_slice_in_dim(V_ext, my * hq, hq, axis=2)
    Kh = K.transpose(0, 2, 1, 3).reshape(B * hq, Skv, DH).astype(bf)
    Vh = V.transpose(0, 2, 1, 3).reshape(B * hq, Skv, DH).astype(bf)

    import numpy as np
    qb = (np.arange(Sq) // BLK)[:, None]
    kb = (np.arange(Skv) // BLK)[None, :]
    mask = (qb == kb) | (kb == 0) | ((qb + kb) % 3 == 0)
    bias = jnp.asarray(np.where(mask, 0.0, -1e9).astype(np.float32))

    out = _fused_attn_allreduce(
        x.reshape(B * Sq, E), (Wq * 0.125).astype(bf),
        Kh, Vh, Wo.astype(bf), bias, B, Sq,
    )
    return out.reshape(B, Sq, E)
